# baseline (speedup 1.0000x reference)
import functools

import jax
import jax.numpy as jnp
from jax import lax
from jax.experimental import pallas as pl
from jax.experimental.pallas import tpu as pltpu
from jax.experimental.pallas import tpu_sc as plsc

N = 10000
E = 160000
D_IN = 256
D_H = 32
G = 64

NC = 2
NS = 16
NW = NC * NS
CHUNK = 200
CPW = 25
EW = CHUNK * CPW
RPT = 640
ACC_ROWS = RPT * NS
ROW_BLK = 2000
N_BLKS = N // ROW_BLK

_mesh = plsc.VectorSubcoreMesh(
    core_axis_name="c", subcore_axis_name="s", num_cores=NC, num_subcores=NS)

_sc_params = pltpu.CompilerParams(use_tc_tiling_on_sc=False)


@functools.partial(
    pl.kernel,
    out_type=jax.ShapeDtypeStruct((NC, ACC_ROWS, D_H), jnp.float32),
    mesh=_mesh,
    compiler_params=_sc_params,
    scratch_types=[
        pltpu.VMEM_SHARED((ACC_ROWS, D_H), jnp.float32),
        pltpu.VMEM((RPT // 2, D_H), jnp.float32),
        pltpu.VMEM((EW,), jnp.int32),
        pltpu.VMEM((CHUNK, D_H), jnp.float32),
    ],
)
def _deg_kernel(edge_index, zeros_hbm, ones_hbm, out, acc, stage, dstv,
                onesv):
    cid = lax.axis_index("c")
    sid = lax.axis_index("s")
    wid = cid * NS + sid
    half = RPT // 2
    pltpu.sync_copy(zeros_hbm, stage)
    for hf in range(2):
        pltpu.sync_copy(stage, acc.at[pl.ds(sid * RPT + hf * half, half)])
    pltpu.sync_copy(ones_hbm, onesv)
    pltpu.sync_copy(edge_index.at[1].at[pl.ds(wid * EW, EW)], dstv)
    plsc.subcore_barrier()

    def body(j, _):
        pltpu.sync_copy(onesv, acc.at[dstv.at[pl.ds(j * CHUNK, CHUNK)]],
                        add=True)
        return 0

    lax.fori_loop(0, CPW, body, 0)
    plsc.subcore_barrier()
    for hf in range(2):
        hr = pl.ds(sid * RPT + hf * half, half)
        pltpu.sync_copy(acc.at[hr], stage)
        pltpu.sync_copy(stage, out.at[cid].at[hr])


@functools.partial(
    pl.kernel,
    out_type=jax.ShapeDtypeStruct((NC, ACC_ROWS, D_H), jnp.float32),
    mesh=_mesh,
    compiler_params=_sc_params,
    scratch_types=[
        pltpu.VMEM_SHARED((ACC_ROWS, D_H), jnp.float32),
        pltpu.VMEM((RPT // 2, D_H), jnp.float32),
        pltpu.VMEM((EW,), jnp.int32),
        pltpu.VMEM((EW,), jnp.int32),
        pltpu.VMEM((CHUNK, D_H), jnp.float32),
        pltpu.VMEM((CHUNK, D_H), jnp.float32),
        pltpu.VMEM((CHUNK, D_H), jnp.float32),
        pltpu.SemaphoreType.DMA,
        pltpu.SemaphoreType.DMA,
        pltpu.SemaphoreType.DMA,
    ],
)
def _mp_kernel(ht, edge_index, zeros_hbm, out, acc, stage, srcv,
               dstv, rows0, rows1, rows2, sem0, sem1, sem2):
    cid = lax.axis_index("c")
    sid = lax.axis_index("s")
    wid = cid * NS + sid
    half = RPT // 2
    pltpu.sync_copy(zeros_hbm, stage)
    for hf in range(2):
        hr = pl.ds(sid * RPT + hf * half, half)
        pltpu.sync_copy(stage, acc.at[hr])
    base = wid * EW
    pltpu.sync_copy(edge_index.at[0].at[pl.ds(base, EW)], srcv)
    pltpu.sync_copy(edge_index.at[1].at[pl.ds(base, EW)], dstv)
    plsc.subcore_barrier()

    rows = (rows0, rows1, rows2)
    sems = (sem0, sem1, sem2)
    pltpu.async_copy(ht.at[srcv.at[pl.ds(0, CHUNK)]], rows0, sem0)
    pltpu.async_copy(ht.at[srcv.at[pl.ds(CHUNK, CHUNK)]], rows1, sem1)

    def body(j3, _):
        for p in range(3):
            j = j3 * 3 + p
            pltpu.make_async_copy(ht.at[srcv.at[pl.ds(0, CHUNK)]],
                                  rows[p], sems[p]).wait()

            @pl.when(j + 2 < CPW)
            def _():
                nxt = pl.ds((j + 2) * CHUNK, CHUNK)
                q = (p + 2) % 3
                pltpu.async_copy(ht.at[srcv.at[nxt]], rows[q], sems[q])

            cur = pl.ds(j * CHUNK, CHUNK)
            pltpu.sync_copy(rows[p], acc.at[dstv.at[cur]], add=True)
        return 0

    lax.fori_loop(0, (CPW - 1) // 3, body, 0)
    pltpu.make_async_copy(ht.at[srcv.at[pl.ds(0, CHUNK)]], rows0,
                          sem0).wait()
    pltpu.sync_copy(rows0, acc.at[dstv.at[pl.ds((CPW - 1) * CHUNK, CHUNK)]],
                    add=True)
    plsc.subcore_barrier()
    for hf in range(2):
        hr = pl.ds(sid * RPT + hf * half, half)
        pltpu.sync_copy(acc.at[hr], stage)
        pltpu.sync_copy(stage, out.at[cid].at[hr])


P_V = ACC_ROWS * D_H // 128
HT_V = N * D_H // 128
NB = N // 4


def _blockdiag4(w, d):
    cols = []
    for k in range(4):
        parts = []
        if k > 0:
            parts.append(jnp.zeros((k * d, D_H), jnp.float32))
        parts.append(w)
        if k < 3:
            parts.append(jnp.zeros(((3 - k) * d, D_H), jnp.float32))
        cols.append(jnp.concatenate(parts, axis=0))
    return jnp.concatenate(cols, axis=1)


def _tile4(v):
    return jnp.concatenate([v, v, v, v], axis=1)


def _layer1_body(x4_ref, w1_ref, h1_ref):
    w1b = _blockdiag4(w1_ref[...], D_IN)
    h1_ref[...] = jnp.dot(x4_ref[...], w1b,
                          preferred_element_type=jnp.float32)


def _scale1_body(degp_ref, h1_ref, ht_ref, dinv_ref):
    deg = degp_ref[0] + degp_ref[1] + 1.0
    dinv = lax.rsqrt(deg)[:HT_V]
    dinv_ref[...] = dinv
    ht_ref[...] = h1_ref[...] * dinv


def _layer2_body(p_ref, ht1_ref, dinv_ref, b1_ref, w2_ref, ht2_ref):
    p = p_ref[0][:HT_V] + p_ref[1][:HT_V]
    dinv = dinv_ref[...]
    out1 = dinv * (p + ht1_ref[...]) + _tile4(b1_ref[...])
    r = jnp.maximum(out1, 0.0)
    w2b = _blockdiag4(w2_ref[...], D_H)
    h2 = jnp.dot(r, w2b, preferred_element_type=jnp.float32)
    ht2_ref[...] = h2 * dinv


def _final_body(p_ref, ht2_ref, dinv_ref, b2_ref, batch4_ref, wc_ref, bc_ref,
                out_ref):
    p = p_ref[0][:HT_V] + p_ref[1][:HT_V]
    out2 = dinv_ref[...] * (p + ht2_ref[...]) + _tile4(b2_ref[...])
    pooled = jnp.zeros((G, D_H), jnp.float32)
    counts = jnp.zeros((G, 1), jnp.float32)
    gids = lax.broadcasted_iota(jnp.int32, (NB, G), 1)
    for k in range(4):
        bk = batch4_ref[:, k:k + 1]
        ohk = (gids == jnp.broadcast_to(bk, (NB, G))).astype(jnp.float32)
        pk = lax.dot_general(ohk, out2, (((0,), (0,)), ((), ())),
                             preferred_element_type=jnp.float32)
        pooled = pooled + pk[:, k * D_H:(k + 1) * D_H]
        counts = counts + jnp.sum(ohk, axis=0, keepdims=True).T
    pooled = pooled / jnp.maximum(counts, 1.0)
    logits = (jnp.dot(pooled, wc_ref[...], preferred_element_type=jnp.float32)
              + bc_ref[...])
    m = jnp.max(logits, axis=1, keepdims=True)
    lse = jnp.log(jnp.sum(jnp.exp(logits - m), axis=1, keepdims=True)) + m
    out_ref[...] = logits - lse


def kernel(x, edge_index, batch, W1, b1, W2, b2, Wc, bc):
    f32 = jnp.float32
    zeros_rows = jnp.zeros((RPT // 2, D_H), f32)
    ones_rows = jnp.ones((CHUNK, D_H), f32)

    h1v = pl.pallas_call(
        _layer1_body,
        out_shape=jax.ShapeDtypeStruct((HT_V, 128), f32),
    )(x.reshape(NB, 4 * D_IN), W1)

    degp = _deg_kernel(edge_index, zeros_rows, ones_rows)

    htv1, dinvv = pl.pallas_call(
        _scale1_body,
        out_shape=[
            jax.ShapeDtypeStruct((HT_V, 128), f32),
            jax.ShapeDtypeStruct((HT_V, 128), f32),
        ],
    )(degp.reshape(NC, P_V, 128), h1v)

    p1 = _mp_kernel(htv1.reshape(N, D_H), edge_index, zeros_rows)

    htv2 = pl.pallas_call(
        _layer2_body,
        out_shape=jax.ShapeDtypeStruct((HT_V, 128), f32),
    )(p1.reshape(NC, P_V, 128), htv1, dinvv, b1.reshape(1, D_H), W2)

    p2 = _mp_kernel(htv2.reshape(N, D_H), edge_index, zeros_rows)

    out = pl.pallas_call(
        _final_body,
        out_shape=jax.ShapeDtypeStruct((G, 2), f32),
    )(p2.reshape(NC, P_V, 128), htv2, dinvv, b2.reshape(1, D_H),
      batch.reshape(NB, 4), Wc, bc.reshape(1, 2))
    return out


# speedup vs baseline: 1.0234x; 1.0234x over previous
import functools

import jax
import jax.numpy as jnp
from jax import lax
from jax.experimental import pallas as pl
from jax.experimental.pallas import tpu as pltpu
from jax.experimental.pallas import tpu_sc as plsc

N = 10000
E = 160000
D_IN = 256
D_H = 32
G = 64

NC = 2
NS = 16
NW = NC * NS
CHUNK = 200
CPW = 25
EW = CHUNK * CPW
RPT = 640
ACC_ROWS = RPT * NS
ROW_BLK = 2000
N_BLKS = N // ROW_BLK

_mesh = plsc.VectorSubcoreMesh(
    core_axis_name="c", subcore_axis_name="s", num_cores=NC, num_subcores=NS)

_sc_params = pltpu.CompilerParams(use_tc_tiling_on_sc=False)


@functools.partial(
    pl.kernel,
    out_type=jax.ShapeDtypeStruct((NC, ACC_ROWS, D_H), jnp.float32),
    mesh=_mesh,
    compiler_params=_sc_params,
    scratch_types=[
        pltpu.VMEM_SHARED((ACC_ROWS, D_H), jnp.float32),
        pltpu.VMEM((RPT // 2, D_H), jnp.float32),
        pltpu.VMEM((EW,), jnp.int32),
        pltpu.VMEM((CHUNK, D_H), jnp.float32),
        pltpu.SemaphoreType.DMA,
    ],
)
def _deg_kernel(edge_index, zeros_hbm, ones_hbm, out, acc, stage, dstv,
                onesv, sem):
    cid = lax.axis_index("c")
    sid = lax.axis_index("s")
    wid = cid * NS + sid
    half = RPT // 2
    pltpu.sync_copy(zeros_hbm, stage)
    for hf in range(2):
        pltpu.sync_copy(stage, acc.at[pl.ds(sid * RPT + hf * half, half)])
    pltpu.sync_copy(ones_hbm, onesv)
    pltpu.sync_copy(edge_index.at[1].at[pl.ds(wid * EW, EW)], dstv)
    plsc.subcore_barrier()

    def body(j, _):
        pltpu.async_copy(onesv, acc.at[dstv.at[pl.ds(j * CHUNK, CHUNK)]],
                         sem, add=True)
        return 0

    lax.fori_loop(0, CPW, body, 0)

    def drain(j, _):
        pltpu.make_async_copy(
            onesv, acc.at[dstv.at[pl.ds(0, CHUNK)]], sem).wait()
        return 0

    lax.fori_loop(0, CPW, drain, 0)
    plsc.subcore_barrier()
    for hf in range(2):
        hr = pl.ds(sid * RPT + hf * half, half)
        pltpu.sync_copy(acc.at[hr], stage)
        pltpu.sync_copy(stage, out.at[cid].at[hr])


@functools.partial(
    pl.kernel,
    out_type=jax.ShapeDtypeStruct((NC, ACC_ROWS, D_H), jnp.float32),
    mesh=_mesh,
    compiler_params=_sc_params,
    scratch_types=[
        pltpu.VMEM_SHARED((ACC_ROWS, D_H), jnp.float32),
        pltpu.VMEM((RPT // 2, D_H), jnp.float32),
        pltpu.VMEM((EW,), jnp.int32),
        pltpu.VMEM((EW,), jnp.int32),
        pltpu.VMEM((CHUNK, D_H), jnp.float32),
        pltpu.VMEM((CHUNK, D_H), jnp.float32),
        pltpu.VMEM((CHUNK, D_H), jnp.float32),
        pltpu.SemaphoreType.DMA,
        pltpu.SemaphoreType.DMA,
        pltpu.SemaphoreType.DMA,
    ],
)
def _mp_kernel(ht, edge_index, zeros_hbm, out, acc, stage, srcv,
               dstv, rows0, rows1, rows2, sem0, sem1, sem2):
    cid = lax.axis_index("c")
    sid = lax.axis_index("s")
    wid = cid * NS + sid
    half = RPT // 2
    pltpu.sync_copy(zeros_hbm, stage)
    for hf in range(2):
        hr = pl.ds(sid * RPT + hf * half, half)
        pltpu.sync_copy(stage, acc.at[hr])
    base = wid * EW
    pltpu.sync_copy(edge_index.at[0].at[pl.ds(base, EW)], srcv)
    pltpu.sync_copy(edge_index.at[1].at[pl.ds(base, EW)], dstv)
    plsc.subcore_barrier()

    rows = (rows0, rows1, rows2)
    sems = (sem0, sem1, sem2)
    pltpu.async_copy(ht.at[srcv.at[pl.ds(0, CHUNK)]], rows0, sem0)
    pltpu.async_copy(ht.at[srcv.at[pl.ds(CHUNK, CHUNK)]], rows1, sem1)

    def body(j3, _):
        for p in range(3):
            j = j3 * 3 + p
            pltpu.make_async_copy(ht.at[srcv.at[pl.ds(0, CHUNK)]],
                                  rows[p], sems[p]).wait()

            @pl.when(j + 2 < CPW)
            def _():
                nxt = pl.ds((j + 2) * CHUNK, CHUNK)
                q = (p + 2) % 3
                pltpu.async_copy(ht.at[srcv.at[nxt]], rows[q], sems[q])

            cur = pl.ds(j * CHUNK, CHUNK)
            pltpu.sync_copy(rows[p], acc.at[dstv.at[cur]], add=True)
        return 0

    lax.fori_loop(0, (CPW - 1) // 3, body, 0)
    pltpu.make_async_copy(ht.at[srcv.at[pl.ds(0, CHUNK)]], rows0,
                          sem0).wait()
    pltpu.sync_copy(rows0, acc.at[dstv.at[pl.ds((CPW - 1) * CHUNK, CHUNK)]],
                    add=True)
    plsc.subcore_barrier()
    for hf in range(2):
        hr = pl.ds(sid * RPT + hf * half, half)
        pltpu.sync_copy(acc.at[hr], stage)
        pltpu.sync_copy(stage, out.at[cid].at[hr])


P_V = ACC_ROWS * D_H // 128
HT_V = N * D_H // 128
NB = N // 4


def _blockdiag4(w, d):
    cols = []
    for k in range(4):
        parts = []
        if k > 0:
            parts.append(jnp.zeros((k * d, D_H), jnp.float32))
        parts.append(w)
        if k < 3:
            parts.append(jnp.zeros(((3 - k) * d, D_H), jnp.float32))
        cols.append(jnp.concatenate(parts, axis=0))
    return jnp.concatenate(cols, axis=1)


def _tile4(v):
    return jnp.concatenate([v, v, v, v], axis=1)


def _layer1_body(x4_ref, w1_ref, h1_ref):
    w1b = _blockdiag4(w1_ref[...], D_IN)
    h1_ref[...] = jnp.dot(x4_ref[...], w1b,
                          preferred_element_type=jnp.float32)


def _scale1_body(degp_ref, h1_ref, ht_ref, dinv_ref):
    deg = degp_ref[0] + degp_ref[1] + 1.0
    dinv = lax.rsqrt(deg)[:HT_V]
    dinv_ref[...] = dinv
    ht_ref[...] = h1_ref[...] * dinv


def _layer2_body(p_ref, ht1_ref, dinv_ref, b1_ref, w2_ref, ht2_ref):
    p = p_ref[0][:HT_V] + p_ref[1][:HT_V]
    dinv = dinv_ref[...]
    out1 = dinv * (p + ht1_ref[...]) + _tile4(b1_ref[...])
    r = jnp.maximum(out1, 0.0)
    w2b = _blockdiag4(w2_ref[...], D_H)
    h2 = jnp.dot(r, w2b, preferred_element_type=jnp.float32)
    ht2_ref[...] = h2 * dinv


def _final_body(p_ref, ht2_ref, dinv_ref, b2_ref, batcht_ref, wc_ref, bc_ref,
                out_ref):
    p = p_ref[0][:HT_V] + p_ref[1][:HT_V]
    out2 = dinv_ref[...] * (p + ht2_ref[...]) + _tile4(b2_ref[...])
    pooled = jnp.zeros((G, D_H), jnp.float32)
    counts = jnp.zeros((G, 1), jnp.float32)
    gids = lax.broadcasted_iota(jnp.int32, (G, NB), 0)
    for k in range(4):
        bk = batcht_ref[k:k + 1, :]
        ohk = (gids == jnp.broadcast_to(bk, (G, NB))).astype(jnp.float32)
        pk = jnp.dot(ohk, out2, preferred_element_type=jnp.float32)
        pooled = pooled + pk[:, k * D_H:(k + 1) * D_H]
        counts = counts + jnp.sum(ohk, axis=1, keepdims=True)
    pooled = pooled / jnp.maximum(counts, 1.0)
    logits = (jnp.dot(pooled, wc_ref[...], preferred_element_type=jnp.float32)
              + bc_ref[...])
    m = jnp.max(logits, axis=1, keepdims=True)
    lse = jnp.log(jnp.sum(jnp.exp(logits - m), axis=1, keepdims=True)) + m
    out_ref[...] = logits - lse


def kernel(x, edge_index, batch, W1, b1, W2, b2, Wc, bc):
    f32 = jnp.float32
    zeros_rows = jnp.zeros((RPT // 2, D_H), f32)
    ones_rows = jnp.ones((CHUNK, D_H), f32)

    h1v = pl.pallas_call(
        _layer1_body,
        out_shape=jax.ShapeDtypeStruct((HT_V, 128), f32),
    )(x.reshape(NB, 4 * D_IN), W1)

    degp = _deg_kernel(edge_index, zeros_rows, ones_rows)

    htv1, dinvv = pl.pallas_call(
        _scale1_body,
        out_shape=[
            jax.ShapeDtypeStruct((HT_V, 128), f32),
            jax.ShapeDtypeStruct((HT_V, 128), f32),
        ],
    )(degp.reshape(NC, P_V, 128), h1v)

    p1 = _mp_kernel(htv1.reshape(N, D_H), edge_index, zeros_rows)

    htv2 = pl.pallas_call(
        _layer2_body,
        out_shape=jax.ShapeDtypeStruct((HT_V, 128), f32),
    )(p1.reshape(NC, P_V, 128), htv1, dinvv, b1.reshape(1, D_H), W2)

    p2 = _mp_kernel(htv2.reshape(N, D_H), edge_index, zeros_rows)

    out = pl.pallas_call(
        _final_body,
        out_shape=jax.ShapeDtypeStruct((G, 2), f32),
    )(p2.reshape(NC, P_V, 128), htv2, dinvv, b2.reshape(1, D_H),
      batch.reshape(NB, 4).T, Wc, bc.reshape(1, 2))
    return out


# speedup vs baseline: 1.0747x; 1.0501x over previous
import functools

import jax
import jax.numpy as jnp
from jax import lax
from jax.experimental import pallas as pl
from jax.experimental.pallas import tpu as pltpu
from jax.experimental.pallas import tpu_sc as plsc

N = 10000
E = 160000
D_IN = 256
D_H = 32
G = 64

NC = 2
NS = 16
NW = NC * NS
CHUNK = 200
CPW = 25
EW = CHUNK * CPW
RPT = 640
ACC_ROWS = RPT * NS
ROW_BLK = 2000
N_BLKS = N // ROW_BLK

_mesh = plsc.VectorSubcoreMesh(
    core_axis_name="c", subcore_axis_name="s", num_cores=NC, num_subcores=NS)

_sc_params = pltpu.CompilerParams(use_tc_tiling_on_sc=False)


@functools.partial(
    pl.kernel,
    out_type=jax.ShapeDtypeStruct((NC, ACC_ROWS, D_H), jnp.float32),
    mesh=_mesh,
    compiler_params=_sc_params,
    scratch_types=[
        pltpu.VMEM_SHARED((ACC_ROWS, D_H), jnp.float32),
        pltpu.VMEM((RPT // 2, D_H), jnp.float32),
        pltpu.VMEM((EW,), jnp.int32),
        pltpu.VMEM((CHUNK, D_H), jnp.float32),
    ],
)
def _deg_kernel(edge_index, zeros_hbm, ones_hbm, out, acc, stage, dstv,
                onesv):
    cid = lax.axis_index("c")
    sid = lax.axis_index("s")
    wid = cid * NS + sid
    half = RPT // 2
    pltpu.sync_copy(zeros_hbm, stage)
    for hf in range(2):
        pltpu.sync_copy(stage, acc.at[pl.ds(sid * RPT + hf * half, half)])
    pltpu.sync_copy(ones_hbm, onesv)
    pltpu.sync_copy(edge_index.at[1].at[pl.ds(wid * EW, EW)], dstv)
    plsc.subcore_barrier()

    def body(j, _):
        pltpu.sync_copy(onesv, acc.at[dstv.at[pl.ds(j * CHUNK, CHUNK)]],
                        add=True)
        return 0

    lax.fori_loop(0, CPW, body, 0)
    plsc.subcore_barrier()
    for hf in range(2):
        hr = pl.ds(sid * RPT + hf * half, half)
        pltpu.sync_copy(acc.at[hr], stage)
        pltpu.sync_copy(stage, out.at[cid].at[hr])


@functools.partial(
    pl.kernel,
    out_type=jax.ShapeDtypeStruct((NC, ACC_ROWS, D_H), jnp.float32),
    mesh=_mesh,
    compiler_params=_sc_params,
    scratch_types=[
        pltpu.VMEM_SHARED((ACC_ROWS, D_H), jnp.float32),
        pltpu.VMEM((RPT // 2, D_H), jnp.float32),
        pltpu.VMEM((EW,), jnp.int32),
        pltpu.VMEM((EW,), jnp.int32),
        pltpu.VMEM((CHUNK, D_H), jnp.float32),
        pltpu.VMEM((CHUNK, D_H), jnp.float32),
        pltpu.VMEM((CHUNK, D_H), jnp.float32),
        pltpu.VMEM((CHUNK, D_H), jnp.float32),
        pltpu.SemaphoreType.DMA,
        pltpu.SemaphoreType.DMA,
        pltpu.SemaphoreType.DMA,
        pltpu.SemaphoreType.DMA,
    ],
)
def _mp_kernel(ht, edge_index, zeros_hbm, out, acc, stage, srcv,
               dstv, rows0, rows1, rows2, rows3, sem0, sem1, sem2, sem3):
    cid = lax.axis_index("c")
    sid = lax.axis_index("s")
    wid = cid * NS + sid
    half = RPT // 2
    pltpu.sync_copy(zeros_hbm, stage)
    for hf in range(2):
        hr = pl.ds(sid * RPT + hf * half, half)
        pltpu.sync_copy(stage, acc.at[hr])
    base = wid * EW
    pltpu.sync_copy(edge_index.at[0].at[pl.ds(base, EW)], srcv)
    pltpu.sync_copy(edge_index.at[1].at[pl.ds(base, EW)], dstv)
    plsc.subcore_barrier()

    rows = (rows0, rows1, rows2, rows3)
    sems = (sem0, sem1, sem2, sem3)
    pltpu.async_copy(ht.at[srcv.at[pl.ds(0, CHUNK)]], rows0, sem0)
    pltpu.async_copy(ht.at[srcv.at[pl.ds(CHUNK, CHUNK)]], rows1, sem1)
    pltpu.async_copy(ht.at[srcv.at[pl.ds(2 * CHUNK, CHUNK)]], rows2, sem2)

    def body(j4, _):
        for p in range(4):
            j = j4 * 4 + p
            pltpu.make_async_copy(ht.at[srcv.at[pl.ds(0, CHUNK)]],
                                  rows[p], sems[p]).wait()

            @pl.when(j + 3 < CPW)
            def _():
                nxt = pl.ds((j + 3) * CHUNK, CHUNK)
                q = (p + 3) % 4
                pltpu.async_copy(ht.at[srcv.at[nxt]], rows[q], sems[q])

            cur = pl.ds(j * CHUNK, CHUNK)
            pltpu.sync_copy(rows[p], acc.at[dstv.at[cur]], add=True)
        return 0

    lax.fori_loop(0, (CPW - 1) // 4, body, 0)
    pltpu.make_async_copy(ht.at[srcv.at[pl.ds(0, CHUNK)]], rows0,
                          sem0).wait()
    pltpu.sync_copy(rows0, acc.at[dstv.at[pl.ds((CPW - 1) * CHUNK, CHUNK)]],
                    add=True)
    plsc.subcore_barrier()
    for hf in range(2):
        hr = pl.ds(sid * RPT + hf * half, half)
        pltpu.sync_copy(acc.at[hr], stage)
        pltpu.sync_copy(stage, out.at[cid].at[hr])


P_V = ACC_ROWS * D_H // 128
HT_V = N * D_H // 128
NB = N // 4


def _blockdiag4(w, d):
    cols = []
    for k in range(4):
        parts = []
        if k > 0:
            parts.append(jnp.zeros((k * d, D_H), jnp.float32))
        parts.append(w)
        if k < 3:
            parts.append(jnp.zeros(((3 - k) * d, D_H), jnp.float32))
        cols.append(jnp.concatenate(parts, axis=0))
    return jnp.concatenate(cols, axis=1)


def _tile4(v):
    return jnp.concatenate([v, v, v, v], axis=1)


def _layer1_body(x4_ref, w1_ref, h1_ref):
    w1b = _blockdiag4(w1_ref[...], D_IN)
    h1_ref[...] = jnp.dot(x4_ref[...], w1b,
                          preferred_element_type=jnp.float32)


def _scale1_body(degp_ref, h1_ref, ht_ref, dinv_ref):
    deg = degp_ref[0] + degp_ref[1] + 1.0
    dinv = lax.rsqrt(deg)[:HT_V]
    dinv_ref[...] = dinv
    ht_ref[...] = h1_ref[...] * dinv


def _layer2_body(p_ref, ht1_ref, dinv_ref, b1_ref, w2_ref, ht2_ref):
    p = p_ref[0][:HT_V] + p_ref[1][:HT_V]
    dinv = dinv_ref[...]
    out1 = dinv * (p + ht1_ref[...]) + _tile4(b1_ref[...])
    r = jnp.maximum(out1, 0.0)
    w2b = _blockdiag4(w2_ref[...], D_H)
    h2 = jnp.dot(r, w2b, preferred_element_type=jnp.float32)
    ht2_ref[...] = h2 * dinv


def _final_body(p_ref, ht2_ref, dinv_ref, b2_ref, batcht_ref, wc_ref, bc_ref,
                out_ref):
    p = p_ref[0][:HT_V] + p_ref[1][:HT_V]
    out2 = dinv_ref[...] * (p + ht2_ref[...]) + _tile4(b2_ref[...])
    pooled = jnp.zeros((G, D_H), jnp.float32)
    counts = jnp.zeros((G, 1), jnp.float32)
    gids = lax.broadcasted_iota(jnp.int32, (G, NB), 0)
    for k in range(4):
        bk = batcht_ref[k:k + 1, :]
        ohk = (gids == jnp.broadcast_to(bk, (G, NB))).astype(jnp.float32)
        pk = jnp.dot(ohk, out2, preferred_element_type=jnp.float32)
        pooled = pooled + pk[:, k * D_H:(k + 1) * D_H]
        counts = counts + jnp.sum(ohk, axis=1, keepdims=True)
    pooled = pooled / jnp.maximum(counts, 1.0)
    logits = (jnp.dot(pooled, wc_ref[...], preferred_element_type=jnp.float32)
              + bc_ref[...])
    m = jnp.max(logits, axis=1, keepdims=True)
    lse = jnp.log(jnp.sum(jnp.exp(logits - m), axis=1, keepdims=True)) + m
    out_ref[...] = logits - lse


def kernel(x, edge_index, batch, W1, b1, W2, b2, Wc, bc):
    f32 = jnp.float32
    zeros_rows = jnp.zeros((RPT // 2, D_H), f32)
    ones_rows = jnp.ones((CHUNK, D_H), f32)

    h1v = pl.pallas_call(
        _layer1_body,
        out_shape=jax.ShapeDtypeStruct((HT_V, 128), f32),
    )(x.reshape(NB, 4 * D_IN), W1)

    degp = _deg_kernel(edge_index, zeros_rows, ones_rows)

    htv1, dinvv = pl.pallas_call(
        _scale1_body,
        out_shape=[
            jax.ShapeDtypeStruct((HT_V, 128), f32),
            jax.ShapeDtypeStruct((HT_V, 128), f32),
        ],
    )(degp.reshape(NC, P_V, 128), h1v)

    p1 = _mp_kernel(htv1.reshape(N, D_H), edge_index, zeros_rows)

    htv2 = pl.pallas_call(
        _layer2_body,
        out_shape=jax.ShapeDtypeStruct((HT_V, 128), f32),
    )(p1.reshape(NC, P_V, 128), htv1, dinvv, b1.reshape(1, D_H), W2)

    p2 = _mp_kernel(htv2.reshape(N, D_H), edge_index, zeros_rows)

    out = pl.pallas_call(
        _final_body,
        out_shape=jax.ShapeDtypeStruct((G, 2), f32),
    )(p2.reshape(NC, P_V, 128), htv2, dinvv, b2.reshape(1, D_H),
      batch.reshape(NB, 4).T, Wc, bc.reshape(1, 2))
    return out
